# Initial kernel scaffold; baseline (speedup 1.0000x reference)
import jax
import jax.numpy as jnp
from jax.experimental import pallas as pl


def kernel(x, edge_index, batch, W1, b1, W2, b2):
    raise NotImplementedError("write your pallas kernel here")


# speedup vs baseline: 9.0932x; 9.0932x over previous
import functools

import jax
import jax.numpy as jnp
from jax import lax
from jax.experimental import pallas as pl
from jax.experimental.pallas import tpu as pltpu
from jax.experimental.pallas import tpu_sc as plsc

N = 10000
E = 160000
G = 64
DIN = 256
DH = 512
NC = 2
NS = 16
NW = NC * NS
EPW = E // NW
EPT = E // NS
CW = 64
NCH1 = DIN // CW
NCH2 = DH // CW
BATCH = 100
NBATCH = EPT // BATCH
NPAD = 10240
SPT = NPAD // NS
ZB = 40

_mesh = plsc.VectorSubcoreMesh(
    core_axis_name="c", subcore_axis_name="s", num_cores=NC, num_subcores=NS)

def _zeros16():
    return jnp.zeros((16,), jnp.float32)


DB = 100
NDB = EPW // DB


def _deg_body(dst_hbm, out_hbm, dst_v, ones_b, acc):
    cid = lax.axis_index("c")
    sid = lax.axis_index("s")
    row0 = sid * SPT

    def fill(v):
        def fb(r, _):
            ones_b[r, pl.ds(0, 16)] = jnp.full((16,), v, jnp.float32)
            return 0
        lax.fori_loop(0, DB, fb, 0)

    fill(0.0)
    for k in range(SPT // ZB):
        pltpu.sync_copy(ones_b.at[pl.ds(0, ZB)],
                        acc.at[pl.ds(row0 + k * ZB, ZB)])
    fill(1.0)

    pltpu.sync_copy(dst_hbm.at[cid * NS + sid], dst_v)
    plsc.subcore_barrier()

    def sbody(j, _):
        pltpu.sync_copy(ones_b, acc.at[dst_v.at[j]], add=True)
        return 0
    lax.fori_loop(0, NDB, sbody, 0)

    plsc.subcore_barrier()
    pltpu.sync_copy(acc.at[pl.ds(row0, SPT)],
                    out_hbm.at[cid, pl.ds(row0, SPT)])


def _deg_call(dst_deg):
    f = pl.kernel(
        _deg_body,
        out_type=jax.ShapeDtypeStruct((NC, NPAD, 16), jnp.float32),
        mesh=_mesh,
        scratch_types=[
            pltpu.VMEM((NDB, DB), jnp.int32),
            pltpu.VMEM((DB, 16), jnp.float32),
            pltpu.VMEM_SHARED((NPAD, 16), jnp.float32),
        ],
        compiler_params=pltpu.CompilerParams(use_tc_tiling_on_sc=False),
    )
    return f(dst_deg)


def _make_edge_body(nchunk):
    cps = nchunk // NC

    def body(tab_hbm, srcadj_hbm, dst_hbm, out_hbm,
             src_v, dst_v, buf, acc, sem0, sem1):
        cid = lax.axis_index("c")
        sid = lax.axis_index("s")
        row0 = sid * SPT

        pltpu.sync_copy(dst_hbm.at[sid], dst_v)

        def zero_acc():
            nv = CW // 16
            def zb(i, _):
                buf[0, i // nv, pl.ds((i % nv) * 16, 16)] = _zeros16()
                return 0
            lax.fori_loop(0, BATCH * nv, zb, 0)
            for k in range(SPT // ZB):
                pltpu.sync_copy(buf.at[0, pl.ds(0, ZB)],
                                acc.at[pl.ds(row0 + k * ZB, ZB)])

        zero_acc()
        sems = (sem0, sem1)

        for c in range(cps):
            ck = cid * cps + c
            pltpu.sync_copy(srcadj_hbm.at[ck * NS + sid], src_v)
            plsc.subcore_barrier()

            def g_start(j, b):
                pltpu.async_copy(tab_hbm.at[src_v.at[j]], buf.at[b], sems[b])

            def g_wait(j, b):
                pltpu.make_async_copy(
                    tab_hbm.at[src_v.at[j]], buf.at[b], sems[b]).wait()

            g_start(0, 0)
            g_start(1, 1)

            def pipe(g2, _):
                for b in range(2):
                    j = g2 * 2 + b
                    g_wait(j, b)
                    pltpu.sync_copy(buf.at[b], acc.at[dst_v.at[j]], add=True)

                    @pl.when(j + 2 < NBATCH)
                    def _():
                        g_start(j + 2, b)
                return 0

            lax.fori_loop(0, NBATCH // 2, pipe, 0)
            plsc.subcore_barrier()

            pltpu.sync_copy(acc.at[pl.ds(row0, SPT)],
                            out_hbm.at[ck, pl.ds(row0, SPT)])
            if c + 1 < cps:
                zero_acc()

    return body


def _edge_call(nchunk, tab, srcadj, dst_r):
    f = pl.kernel(
        _make_edge_body(nchunk),
        out_type=jax.ShapeDtypeStruct((nchunk, NPAD, CW), jnp.float32),
        mesh=_mesh,
        scratch_types=[
            pltpu.VMEM((NBATCH, BATCH), jnp.int32),
            pltpu.VMEM((NBATCH, BATCH), jnp.int32),
            pltpu.VMEM((2, BATCH, CW), jnp.float32),
            pltpu.VMEM_SHARED((NPAD, CW), jnp.float32),
            pltpu.SemaphoreType.DMA,
            pltpu.SemaphoreType.DMA,
        ],
        compiler_params=pltpu.CompilerParams(use_tc_tiling_on_sc=False),
    )
    return f(tab, srcadj, dst_r)


def _scale_body(p0_ref, p1_ref, x_ref, xs_ref, dinv_ref, dinv2_ref):
    deg = p0_ref[...] + p1_ref[...] + 1.0
    dinv = lax.rsqrt(deg)
    dinv_ref[...] = dinv
    dinv2_ref[...] = 1.0 / deg
    xs = x_ref[...] * dinv
    for c in range(NCH1):
        xs_ref[c] = xs[:, c * CW:(c + 1) * CW]


def _scale_call(p0, p1, x):
    return pl.pallas_call(
        _scale_body,
        out_shape=(
            jax.ShapeDtypeStruct((NCH1, N, CW), jnp.float32),
            jax.ShapeDtypeStruct((N, 1), jnp.float32),
            jax.ShapeDtypeStruct((N, 1), jnp.float32),
        ),
    )(p0, p1, x)


BN = 400
NBLK = N // BN

_POOL_DNUMS = (((0,), (0,)), ((), ()))


def _onehot(bt):
    return (bt == lax.broadcasted_iota(jnp.int32, (BN, G), 1)).astype(
        jnp.float32)


def _layer1_body(s_ref, x_ref, dinv_ref, dinv2_ref, w_ref, b_ref, bt_ref,
                 out1_ref, ys_ref, pool_ref, cnt_ref, pacc, cacc):
    i = pl.program_id(0)
    dinv = dinv_ref[...]
    z = (jnp.concatenate([s_ref[c] for c in range(NCH1)], axis=1) * dinv
         + x_ref[...] * dinv2_ref[...])
    o = jnp.dot(z, w_ref[...], preferred_element_type=jnp.float32) + b_ref[...]
    o = jnp.maximum(o, 0.0)
    out1_ref[...] = o
    ys = o * dinv
    for c in range(NCH2):
        ys_ref[c] = ys[:, c * CW:(c + 1) * CW]

    oh = _onehot(bt_ref[...])

    @pl.when(i == 0)
    def _():
        pacc[...] = jnp.zeros_like(pacc)
        cacc[...] = jnp.zeros_like(cacc)

    pacc[...] += lax.dot_general(oh, o, _POOL_DNUMS,
                                 preferred_element_type=jnp.float32)
    cacc[...] += lax.dot_general(oh, jnp.ones((BN, 1), jnp.float32),
                                 _POOL_DNUMS,
                                 preferred_element_type=jnp.float32)

    @pl.when(i == NBLK - 1)
    def _():
        pool_ref[...] = pacc[...] / jnp.maximum(cacc[...], 1.0)
        cnt_ref[...] = cacc[...]


def _layer1_call(s1, x, dinv, dinv2, W1, b1, bt):
    return pl.pallas_call(
        _layer1_body,
        grid=(NBLK,),
        in_specs=[
            pl.BlockSpec((NCH1, BN, CW), lambda i: (0, i, 0)),
            pl.BlockSpec((BN, DIN), lambda i: (i, 0)),
            pl.BlockSpec((BN, 1), lambda i: (i, 0)),
            pl.BlockSpec((BN, 1), lambda i: (i, 0)),
            pl.BlockSpec((DIN, DH), lambda i: (0, 0)),
            pl.BlockSpec((1, DH), lambda i: (0, 0)),
            pl.BlockSpec((BN, 1), lambda i: (i, 0)),
        ],
        out_specs=(
            pl.BlockSpec((BN, DH), lambda i: (i, 0)),
            pl.BlockSpec((NCH2, BN, CW), lambda i: (0, i, 0)),
            pl.BlockSpec((G, DH), lambda i: (0, 0)),
            pl.BlockSpec((G, 1), lambda i: (0, 0)),
        ),
        out_shape=(
            jax.ShapeDtypeStruct((N, DH), jnp.float32),
            jax.ShapeDtypeStruct((NCH2, N, CW), jnp.float32),
            jax.ShapeDtypeStruct((G, DH), jnp.float32),
            jax.ShapeDtypeStruct((G, 1), jnp.float32),
        ),
        scratch_shapes=[
            pltpu.VMEM((G, DH), jnp.float32),
            pltpu.VMEM((G, 1), jnp.float32),
        ],
    )(s1, x, dinv, dinv2, W1, b1, bt)


def _layer2_body(s_ref, o1_ref, dinv_ref, dinv2_ref, w_ref, b_ref, bt_ref,
                 cnt_ref, pool_ref, pacc):
    i = pl.program_id(0)
    z = (jnp.concatenate([s_ref[c] for c in range(NCH2)], axis=1) * dinv_ref[...]
         + o1_ref[...] * dinv2_ref[...])
    o = jnp.dot(z, w_ref[...], preferred_element_type=jnp.float32) + b_ref[...]
    o = jnp.maximum(o, 0.0)
    oh = _onehot(bt_ref[...])

    @pl.when(i == 0)
    def _():
        pacc[...] = jnp.zeros_like(pacc)

    pacc[...] += lax.dot_general(oh, o, _POOL_DNUMS,
                                 preferred_element_type=jnp.float32)

    @pl.when(i == NBLK - 1)
    def _():
        pool_ref[...] = pacc[...] / jnp.maximum(cnt_ref[...], 1.0)


def _layer2_call(s2, out1, dinv, dinv2, W2, b2, bt, cnt):
    return pl.pallas_call(
        _layer2_body,
        grid=(NBLK,),
        in_specs=[
            pl.BlockSpec((NCH2, BN, CW), lambda i: (0, i, 0)),
            pl.BlockSpec((BN, DH), lambda i: (i, 0)),
            pl.BlockSpec((BN, 1), lambda i: (i, 0)),
            pl.BlockSpec((BN, 1), lambda i: (i, 0)),
            pl.BlockSpec((DH, DH), lambda i: (0, 0)),
            pl.BlockSpec((1, DH), lambda i: (0, 0)),
            pl.BlockSpec((BN, 1), lambda i: (i, 0)),
            pl.BlockSpec((G, 1), lambda i: (0, 0)),
        ],
        out_specs=pl.BlockSpec((G, DH), lambda i: (0, 0)),
        out_shape=jax.ShapeDtypeStruct((G, DH), jnp.float32),
        scratch_shapes=[pltpu.VMEM((G, DH), jnp.float32)],
    )(s2, out1, dinv, dinv2, W2, b2, bt, cnt)


def kernel(x, edge_index, batch, W1, b1, W2, b2):
    src = edge_index[0]
    dst = edge_index[1]

    p = _deg_call(dst.reshape(NW, NDB, DB))
    p0 = p[0, :N, 0].reshape(N, 1)
    p1 = p[1, :N, 0].reshape(N, 1)

    xs, dinv, dinv2 = _scale_call(p0, p1, x)

    dst_r = dst.reshape(NS, NBATCH, BATCH)
    off1 = (jnp.arange(NCH1, dtype=jnp.int32) * N)[:, None]
    srcadj1 = (src[None, :] + off1).reshape(NCH1 * NS, NBATCH, BATCH)
    s1 = _edge_call(NCH1, xs.reshape(NCH1 * N, CW), srcadj1, dst_r)[:, :N, :]

    bt = batch.reshape(N, 1)
    out1, ys, pool1, cnt = _layer1_call(s1, x, dinv, dinv2, W1,
                                        b1.reshape(1, DH), bt)

    off2 = (jnp.arange(NCH2, dtype=jnp.int32) * N)[:, None]
    srcadj2 = (src[None, :] + off2).reshape(NCH2 * NS, NBATCH, BATCH)
    s2 = _edge_call(NCH2, ys.reshape(NCH2 * N, CW), srcadj2, dst_r)[:, :N, :]

    pool2 = _layer2_call(s2, out1, dinv, dinv2, W2, b2.reshape(1, DH), bt, cnt)

    return (x, pool1, pool2)
